# baseline (speedup 1.0000x reference)
import jax
import jax.numpy as jnp
from jax.experimental import pallas as pl


def _gcn_kernel(x_ref, w_ref, b_ref, adj_ref, support_ref, hidden_ref, out_ref):
    i = pl.program_id(0)

    @pl.when(i == 0)
    def _():
        support_ref[...] = jnp.dot(
            x_ref[...], w_ref[...], preferred_element_type=jnp.float32
        )

    acc = jnp.dot(
        adj_ref[...], support_ref[...], preferred_element_type=jnp.float32
    )
    acc = acc + b_ref[...]
    out_ref[...] = acc
    hidden_ref[...] = jnp.maximum(acc, 0.0)


def kernel(x, adj, grad_adj, W, b):
    N, din = x.shape
    dout = W.shape[1]

    block_r = min(616, N)
    grid = (pl.cdiv(N, block_r),)

    b2 = b.reshape(1, dout)

    support, hidden, output = pl.pallas_call(
        _gcn_kernel,
        grid=grid,
        in_specs=[
            pl.BlockSpec((N, din), lambda i: (0, 0)),
            pl.BlockSpec((din, dout), lambda i: (0, 0)),
            pl.BlockSpec((1, dout), lambda i: (0, 0)),
            pl.BlockSpec((block_r, N), lambda i: (i, 0)),
        ],
        out_specs=[
            pl.BlockSpec((N, dout), lambda i: (0, 0)),
            pl.BlockSpec((block_r, dout), lambda i: (i, 0)),
            pl.BlockSpec((block_r, dout), lambda i: (i, 0)),
        ],
        out_shape=[
            jax.ShapeDtypeStruct((N, dout), jnp.float32),
            jax.ShapeDtypeStruct((N, dout), jnp.float32),
            jax.ShapeDtypeStruct((N, dout), jnp.float32),
        ],
    )(x, W, b2, adj)

    return (support, hidden, output)


# speedup vs baseline: 1.0227x; 1.0227x over previous
import jax
import jax.numpy as jnp
from jax.experimental import pallas as pl
from jax.experimental.pallas import tpu as pltpu

_BR = 400


def _gcn_kernel(x_ref, w_ref, b_ref, adj_ref, support_ref, hidden_ref,
                out_ref, sup_full):
    i = pl.program_id(0)

    @pl.when(i == 0)
    def _():
        sup_full[...] = jnp.dot(
            x_ref[...], w_ref[...], preferred_element_type=jnp.float32
        )

    support_ref[...] = sup_full[pl.ds(i * _BR, _BR), :]
    acc = jnp.dot(
        adj_ref[...], sup_full[...], preferred_element_type=jnp.float32
    )
    acc = acc + b_ref[...]
    out_ref[...] = acc
    hidden_ref[...] = jnp.maximum(acc, 0.0)


def kernel(x, adj, grad_adj, W, b):
    N, din = x.shape
    dout = W.shape[1]
    grid = (N // _BR,)

    b2 = b.reshape(1, dout)

    support, hidden, output = pl.pallas_call(
        _gcn_kernel,
        grid=grid,
        in_specs=[
            pl.BlockSpec((N, din), lambda i: (0, 0)),
            pl.BlockSpec((din, dout), lambda i: (0, 0)),
            pl.BlockSpec((1, dout), lambda i: (0, 0)),
            pl.BlockSpec((_BR, N), lambda i: (i, 0)),
        ],
        out_specs=[
            pl.BlockSpec((_BR, dout), lambda i: (i, 0)),
            pl.BlockSpec((_BR, dout), lambda i: (i, 0)),
            pl.BlockSpec((_BR, dout), lambda i: (i, 0)),
        ],
        out_shape=[
            jax.ShapeDtypeStruct((N, dout), jnp.float32),
            jax.ShapeDtypeStruct((N, dout), jnp.float32),
            jax.ShapeDtypeStruct((N, dout), jnp.float32),
        ],
        scratch_shapes=[
            pltpu.VMEM((N, dout), jnp.float32),
        ],
    )(x, W, b2, adj)

    return (support, hidden, output)


# speedup vs baseline: 1.0280x; 1.0052x over previous
import jax
import jax.numpy as jnp
from jax.experimental import pallas as pl


def _gcn_kernel(x_ref, w_ref, b_ref, adj_ref, support_ref, hidden_ref, out_ref):
    i = pl.program_id(0)

    @pl.when(i == 0)
    def _():
        support_ref[...] = jnp.dot(
            x_ref[...], w_ref[...], preferred_element_type=jnp.float32
        )

    acc = jnp.dot(
        adj_ref[...], support_ref[...], preferred_element_type=jnp.float32
    )
    acc = acc + b_ref[...]
    out_ref[...] = acc
    hidden_ref[...] = jnp.maximum(acc, 0.0)


def kernel(x, adj, grad_adj, W, b):
    N, din = x.shape
    dout = W.shape[1]

    block_r = 400
    if N % block_r != 0:
        block_r = N
    grid = (N // block_r,)

    b2 = b.reshape(1, dout)

    support, hidden, output = pl.pallas_call(
        _gcn_kernel,
        grid=grid,
        in_specs=[
            pl.BlockSpec((N, din), lambda i: (0, 0)),
            pl.BlockSpec((din, dout), lambda i: (0, 0)),
            pl.BlockSpec((1, dout), lambda i: (0, 0)),
            pl.BlockSpec((block_r, N), lambda i: (i, 0)),
        ],
        out_specs=[
            pl.BlockSpec((N, dout), lambda i: (0, 0)),
            pl.BlockSpec((block_r, dout), lambda i: (i, 0)),
            pl.BlockSpec((block_r, dout), lambda i: (i, 0)),
        ],
        out_shape=[
            jax.ShapeDtypeStruct((N, dout), jnp.float32),
            jax.ShapeDtypeStruct((N, dout), jnp.float32),
            jax.ShapeDtypeStruct((N, dout), jnp.float32),
        ],
    )(x, W, b2, adj)

    return (support, hidden, output)
